# baseline (speedup 1.0000x reference)
import functools

import jax
import jax.numpy as jnp
from jax import lax
from jax.experimental import pallas as pl
from jax.experimental.pallas import tpu as pltpu
from jax.experimental.pallas import tpu_sc as plsc

VOCAB = 1000000
EMBED = 64
EMBEDP = 128
BATCH = 4096
SEQLEN = 200

NUM_CORES = 2
NUM_SUBCORES = 16
NW = NUM_CORES * NUM_SUBCORES
BPW = BATCH // NW
NCHUNK = 2
CHUNK = SEQLEN // NCHUNK
NVREG = EMBED // 16


def _bow_body(idx_hbm, table_hbm, out_hbm, idx_v, buf, out_v, sem):
    wid = lax.axis_index("s") * NUM_CORES + lax.axis_index("c")
    base = wid * BPW

    pltpu.sync_copy(idx_hbm.at[pl.ds(base, BPW)], idx_v)

    def row_body(b, _):
        h0 = pltpu.async_copy(
            table_hbm.at[idx_v.at[b, 0]], buf.at[pl.ds(0, CHUNK)], sem)
        h1 = pltpu.async_copy(
            table_hbm.at[idx_v.at[b, 1]], buf.at[pl.ds(CHUNK, CHUNK)], sem)
        h0.wait()
        h1.wait()

        accs = tuple(buf[0, pl.ds(16 * c, 16)] for c in range(NVREG))

        def red_body(r, accs):
            return tuple(
                jnp.maximum(a, buf[r, pl.ds(16 * c, 16)])
                for c, a in enumerate(accs))

        accs = lax.fori_loop(1, SEQLEN, red_body, accs, unroll=8)
        for c in range(NVREG):
            out_v[b, pl.ds(16 * c, 16)] = accs[c]
        return ()

    lax.fori_loop(0, BPW, row_body, ())

    pltpu.sync_copy(out_v, out_hbm.at[pl.ds(base, BPW)])


@functools.cache
def _bow():
    return functools.partial(
        pl.kernel,
        mesh=plsc.VectorSubcoreMesh(core_axis_name="c", subcore_axis_name="s"),
        out_type=jax.ShapeDtypeStruct((BATCH, EMBED), jnp.float32),
        scratch_types=[
            pltpu.VMEM((BPW, NCHUNK, CHUNK), jnp.int32),
            pltpu.VMEM((SEQLEN, EMBEDP), jnp.float32),
            pltpu.VMEM((BPW, EMBED), jnp.float32),
            pltpu.SemaphoreType.DMA,
        ],
        compiler_params=pltpu.CompilerParams(use_tc_tiling_on_sc=False),
    )(_bow_body)


@jax.jit
def kernel(input, table):
    idx = input.reshape(BATCH, NCHUNK, CHUNK)
    tablep = jnp.pad(table, ((0, 0), (0, EMBEDP - EMBED)))
    return _bow()(idx, tablep)


# speedup vs baseline: 1.2688x; 1.2688x over previous
import functools

import jax
import jax.numpy as jnp
from jax import lax
from jax.experimental import pallas as pl
from jax.experimental.pallas import tpu as pltpu
from jax.experimental.pallas import tpu_sc as plsc

VOCAB = 1000000
EMBED = 64
EMBEDP = 128
BATCH = 4096
SEQLEN = 200

NUM_CORES = 2
NUM_SUBCORES = 16
NW = NUM_CORES * NUM_SUBCORES
BPW = BATCH // NW
CHUNK0 = 104
CHUNK1 = SEQLEN - CHUNK0
NVREG = EMBED // 16

PREP_BLOCK = 4096


def _prep_body(tt_ref, out_ref):
    rows = lax.broadcasted_iota(jnp.int32, (EMBED, EMBEDP), 0)
    cols = lax.broadcasted_iota(jnp.int32, (EMBED, EMBEDP), 1)
    ident = (rows == cols).astype(jnp.float32)
    out_ref[...] = lax.dot_general(
        tt_ref[...], ident,
        dimension_numbers=(((0,), (0,)), ((), ())),
        preferred_element_type=jnp.float32,
    )


def _prep(tt):
    grid = pl.cdiv(VOCAB, PREP_BLOCK)
    return pl.pallas_call(
        _prep_body,
        grid=(grid,),
        in_specs=[pl.BlockSpec((EMBED, PREP_BLOCK), lambda i: (0, i))],
        out_specs=pl.BlockSpec((PREP_BLOCK, EMBEDP), lambda i: (i, 0)),
        out_shape=jax.ShapeDtypeStruct((VOCAB, EMBEDP), jnp.float32),
    )(tt)


def _bow_body(idx_hbm, table_hbm, out_hbm, idx_v, buf, out_v, sem):
    wid = lax.axis_index("s") * NUM_CORES + lax.axis_index("c")
    base = wid * BPW * SEQLEN

    pltpu.sync_copy(idx_hbm.at[pl.ds(base, BPW * SEQLEN)], idx_v)

    def row_body(b, _):
        h0 = pltpu.async_copy(
            table_hbm.at[idx_v.at[pl.ds(b * SEQLEN, CHUNK0)]],
            buf.at[pl.ds(0, CHUNK0)], sem)
        h1 = pltpu.async_copy(
            table_hbm.at[idx_v.at[pl.ds(b * SEQLEN + CHUNK0, CHUNK1)]],
            buf.at[pl.ds(CHUNK0, CHUNK1)], sem)
        h0.wait()
        h1.wait()

        accs = tuple(buf[0, pl.ds(16 * c, 16)] for c in range(NVREG))

        def red_body(r, accs):
            return tuple(
                jnp.maximum(a, buf[r, pl.ds(16 * c, 16)])
                for c, a in enumerate(accs))

        accs = lax.fori_loop(1, SEQLEN, red_body, accs, unroll=8)
        for c in range(NVREG):
            out_v[pl.ds(b * EMBED + 16 * c, 16)] = accs[c]
        return ()

    lax.fori_loop(0, BPW, row_body, ())

    pltpu.sync_copy(out_v, out_hbm.at[pl.ds(wid * BPW * EMBED, BPW * EMBED)])


@functools.cache
def _bow():
    return functools.partial(
        pl.kernel,
        mesh=plsc.VectorSubcoreMesh(core_axis_name="c", subcore_axis_name="s"),
        out_type=jax.ShapeDtypeStruct((BATCH * EMBED,), jnp.float32),
        scratch_types=[
            pltpu.VMEM((BPW * SEQLEN,), jnp.int32),
            pltpu.VMEM((SEQLEN, EMBEDP), jnp.float32),
            pltpu.VMEM((BPW * EMBED,), jnp.float32),
            pltpu.SemaphoreType.DMA,
        ],
        compiler_params=pltpu.CompilerParams(use_tc_tiling_on_sc=True),
    )(_bow_body)


@jax.jit
def kernel(input, table):
    tablep = _prep(table.T)
    idx = input.reshape(BATCH * SEQLEN)
    out = _bow()(idx, tablep)
    return out.reshape(BATCH, EMBED)


# speedup vs baseline: 1.4247x; 1.1228x over previous
import functools

import jax
import jax.numpy as jnp
from jax import lax
from jax.experimental import pallas as pl
from jax.experimental.pallas import tpu as pltpu
from jax.experimental.pallas import tpu_sc as plsc

VOCAB = 1000000
EMBED = 64
EMBEDP = 128
BATCH = 4096
SEQLEN = 200

NUM_CORES = 2
NUM_SUBCORES = 16
NW = NUM_CORES * NUM_SUBCORES
BPW = BATCH // NW
CHUNK0 = 104
CHUNK1 = SEQLEN - CHUNK0
NVREG = EMBED // 16

PREP_BLOCK = 4096


def _prep_body(tt_ref, out_ref):
    rows = lax.broadcasted_iota(jnp.int32, (EMBED, EMBEDP), 0)
    cols = lax.broadcasted_iota(jnp.int32, (EMBED, EMBEDP), 1)
    ident = (rows == cols).astype(jnp.float32)
    out_ref[...] = lax.dot_general(
        tt_ref[...], ident,
        dimension_numbers=(((0,), (0,)), ((), ())),
        preferred_element_type=jnp.float32,
    )


def _prep(tt):
    grid = pl.cdiv(VOCAB, PREP_BLOCK)
    return pl.pallas_call(
        _prep_body,
        grid=(grid,),
        in_specs=[pl.BlockSpec((EMBED, PREP_BLOCK), lambda i: (0, i))],
        out_specs=pl.BlockSpec((PREP_BLOCK, EMBEDP), lambda i: (i, 0)),
        out_shape=jax.ShapeDtypeStruct((VOCAB, EMBEDP), jnp.float32),
    )(tt)


def _bow_body(idx_hbm, table_hbm, out_hbm, idx_v, buf, out_v, sem):
    wid = lax.axis_index("s") * NUM_CORES + lax.axis_index("c")
    base = wid * BPW * SEQLEN

    pltpu.sync_copy(idx_hbm.at[pl.ds(base, BPW * SEQLEN)], idx_v)

    def row_body(b, _):
        h0 = pltpu.async_copy(
            table_hbm.at[idx_v.at[pl.ds(b * SEQLEN, CHUNK0)]],
            buf.at[pl.ds(0, CHUNK0)], sem)
        h1 = pltpu.async_copy(
            table_hbm.at[idx_v.at[pl.ds(b * SEQLEN + CHUNK0, CHUNK1)]],
            buf.at[pl.ds(CHUNK0, CHUNK1)], sem)
        h0.wait()
        h1.wait()

        accs = tuple(buf[0, pl.ds(16 * c, 16)] for c in range(NVREG))

        def red_body(r, accs):
            return tuple(
                jnp.maximum(a, buf[r, pl.ds(16 * c, 16)])
                for c, a in enumerate(accs))

        accs = lax.fori_loop(1, SEQLEN, red_body, accs, unroll=8)
        for c in range(NVREG):
            out_v[pl.ds(b * EMBED + 16 * c, 16)] = accs[c]
        return ()

    lax.fori_loop(0, BPW, row_body, ())

    pltpu.sync_copy(out_v, out_hbm.at[pl.ds(wid * BPW * EMBED, BPW * EMBED)])


@functools.cache
def _bow():
    return functools.partial(
        pl.kernel,
        mesh=plsc.VectorSubcoreMesh(core_axis_name="c", subcore_axis_name="s"),
        out_type=jax.ShapeDtypeStruct((BATCH * EMBED,), jnp.float32),
        scratch_types=[
            pltpu.VMEM((BPW * SEQLEN,), jnp.int32),
            pltpu.VMEM((SEQLEN, EMBED), jnp.float32),
            pltpu.VMEM((BPW * EMBED,), jnp.float32),
            pltpu.SemaphoreType.DMA,
        ],
        compiler_params=pltpu.CompilerParams(use_tc_tiling_on_sc=False),
    )(_bow_body)


@jax.jit
def kernel(input, table):
    tablep = _prep(table.T).reshape(2 * VOCAB, EMBED)
    idx = (input * 2).reshape(BATCH * SEQLEN)
    out = _bow()(idx, tablep)
    return out.reshape(BATCH, EMBED)


# speedup vs baseline: 1.7221x; 1.2088x over previous
import functools

import jax
import jax.numpy as jnp
from jax import lax
from jax.experimental import pallas as pl
from jax.experimental.pallas import tpu as pltpu
from jax.experimental.pallas import tpu_sc as plsc

VOCAB = 1000000
EMBED = 64
EMBEDP = 128
BATCH = 4096
SEQLEN = 200

NUM_CORES = 2
NUM_SUBCORES = 16
NW = NUM_CORES * NUM_SUBCORES
BPW = BATCH // NW
CHUNK0 = 104
CHUNK1 = SEQLEN - CHUNK0
NVREG = EMBED // 16

PREP_BLOCK = 16384


def _prep_body(tt_ref, out_ref):
    rows = lax.broadcasted_iota(jnp.int32, (EMBED, EMBEDP), 0)
    cols = lax.broadcasted_iota(jnp.int32, (EMBED, EMBEDP), 1)
    ident = (rows == cols).astype(jnp.float32)
    out_ref[...] = lax.dot_general(
        tt_ref[...], ident,
        dimension_numbers=(((0,), (0,)), ((), ())),
        preferred_element_type=jnp.float32,
    )


def _prep(tt):
    grid = pl.cdiv(VOCAB, PREP_BLOCK)
    return pl.pallas_call(
        _prep_body,
        grid=(grid,),
        in_specs=[pl.BlockSpec((EMBED, PREP_BLOCK), lambda i: (0, i))],
        out_specs=pl.BlockSpec((PREP_BLOCK, EMBEDP), lambda i: (i, 0)),
        out_shape=jax.ShapeDtypeStruct((VOCAB, EMBEDP), jnp.float32),
    )(tt)


def _bow_body(idx_hbm, table_hbm, out_hbm, idx_v, buf, out_v, sem):
    wid = lax.axis_index("s") * NUM_CORES + lax.axis_index("c")
    base = wid * BPW * SEQLEN

    pltpu.sync_copy(idx_hbm.at[pl.ds(base, BPW * SEQLEN)], idx_v)

    def row_body(b, _):
        h0 = pltpu.async_copy(
            table_hbm.at[idx_v.at[pl.ds(b * SEQLEN, CHUNK0)]],
            buf.at[pl.ds(0, CHUNK0)], sem)
        h1 = pltpu.async_copy(
            table_hbm.at[idx_v.at[pl.ds(b * SEQLEN + CHUNK0, CHUNK1)]],
            buf.at[pl.ds(CHUNK0, CHUNK1)], sem)
        h0.wait()
        h1.wait()

        accs = tuple(buf[0, pl.ds(16 * c, 16)] for c in range(NVREG))

        def red_body(r, accs):
            return tuple(
                jnp.maximum(a, buf[r, pl.ds(16 * c, 16)])
                for c, a in enumerate(accs))

        accs = lax.fori_loop(1, SEQLEN, red_body, accs, unroll=8)
        for c in range(NVREG):
            out_v[pl.ds(b * EMBED + 16 * c, 16)] = accs[c]
        return ()

    lax.fori_loop(0, BPW, row_body, ())

    pltpu.sync_copy(out_v, out_hbm.at[pl.ds(wid * BPW * EMBED, BPW * EMBED)])


@functools.cache
def _bow():
    return functools.partial(
        pl.kernel,
        mesh=plsc.VectorSubcoreMesh(core_axis_name="c", subcore_axis_name="s"),
        out_type=jax.ShapeDtypeStruct((BATCH * EMBED,), jnp.float32),
        scratch_types=[
            pltpu.VMEM((BPW * SEQLEN,), jnp.int32),
            pltpu.VMEM((SEQLEN, EMBED), jnp.float32),
            pltpu.VMEM((BPW * EMBED,), jnp.float32),
            pltpu.SemaphoreType.DMA,
        ],
        compiler_params=pltpu.CompilerParams(use_tc_tiling_on_sc=False),
    )(_bow_body)


@jax.jit
def kernel(input, table):
    tablep = _prep(table.T).reshape(2 * VOCAB, EMBED)
    idx = (input * 2).reshape(BATCH * SEQLEN)
    out = _bow()(idx, tablep)
    return out.reshape(BATCH, EMBED)


# speedup vs baseline: 1.7456x; 1.0137x over previous
import functools

import jax
import jax.numpy as jnp
from jax import lax
from jax.experimental import pallas as pl
from jax.experimental.pallas import tpu as pltpu
from jax.experimental.pallas import tpu_sc as plsc

VOCAB = 1000000
EMBED = 64
EMBEDP = 128
BATCH = 4096
SEQLEN = 200

NUM_CORES = 2
NUM_SUBCORES = 16
NW = NUM_CORES * NUM_SUBCORES
BPW = BATCH // NW
CHUNK0 = 104
CHUNK1 = SEQLEN - CHUNK0
NVREG = EMBED // 16

PREP_BLOCK = 32768


def _prep_body(tt_ref, out_ref):
    rows = lax.broadcasted_iota(jnp.int32, (EMBED, EMBEDP), 0)
    cols = lax.broadcasted_iota(jnp.int32, (EMBED, EMBEDP), 1)
    ident = (rows == cols).astype(jnp.float32)
    out_ref[...] = lax.dot_general(
        tt_ref[...], ident,
        dimension_numbers=(((0,), (0,)), ((), ())),
        preferred_element_type=jnp.float32,
    )


def _prep(tt):
    grid = pl.cdiv(VOCAB, PREP_BLOCK)
    return pl.pallas_call(
        _prep_body,
        grid=(grid,),
        in_specs=[pl.BlockSpec((EMBED, PREP_BLOCK), lambda i: (0, i))],
        out_specs=pl.BlockSpec((PREP_BLOCK, EMBEDP), lambda i: (i, 0)),
        out_shape=jax.ShapeDtypeStruct((VOCAB, EMBEDP), jnp.float32),
    )(tt)


def _bow_body(idx_hbm, table_hbm, out_hbm, idx_v, buf, out_v, sem):
    wid = lax.axis_index("s") * NUM_CORES + lax.axis_index("c")
    base = wid * BPW * SEQLEN

    pltpu.sync_copy(idx_hbm.at[pl.ds(base, BPW * SEQLEN)], idx_v)

    def row_body(b, _):
        h0 = pltpu.async_copy(
            table_hbm.at[idx_v.at[pl.ds(b * SEQLEN, CHUNK0)]],
            buf.at[pl.ds(0, CHUNK0)], sem)
        h1 = pltpu.async_copy(
            table_hbm.at[idx_v.at[pl.ds(b * SEQLEN + CHUNK0, CHUNK1)]],
            buf.at[pl.ds(CHUNK0, CHUNK1)], sem)
        h0.wait()
        h1.wait()

        accs = tuple(buf[0, pl.ds(16 * c, 16)] for c in range(NVREG))

        def red_body(r, accs):
            return tuple(
                jnp.maximum(a, buf[r, pl.ds(16 * c, 16)])
                for c, a in enumerate(accs))

        accs = lax.fori_loop(1, SEQLEN, red_body, accs, unroll=8)
        for c in range(NVREG):
            out_v[pl.ds(b * EMBED + 16 * c, 16)] = accs[c]
        return ()

    lax.fori_loop(0, BPW, row_body, ())

    pltpu.sync_copy(out_v, out_hbm.at[pl.ds(wid * BPW * EMBED, BPW * EMBED)])


@functools.cache
def _bow():
    return functools.partial(
        pl.kernel,
        mesh=plsc.VectorSubcoreMesh(core_axis_name="c", subcore_axis_name="s"),
        out_type=jax.ShapeDtypeStruct((BATCH * EMBED,), jnp.float32),
        scratch_types=[
            pltpu.VMEM((BPW * SEQLEN,), jnp.int32),
            pltpu.VMEM((SEQLEN, EMBED), jnp.float32),
            pltpu.VMEM((BPW * EMBED,), jnp.float32),
            pltpu.SemaphoreType.DMA,
        ],
        compiler_params=pltpu.CompilerParams(use_tc_tiling_on_sc=False),
    )(_bow_body)


@jax.jit
def kernel(input, table):
    tablep = _prep(table.T).reshape(2 * VOCAB, EMBED)
    idx = (input * 2).reshape(BATCH * SEQLEN)
    out = _bow()(idx, tablep)
    return out.reshape(BATCH, EMBED)


# speedup vs baseline: 2.1981x; 1.2592x over previous
import functools

import jax
import jax.numpy as jnp
from jax import lax
from jax.experimental import pallas as pl
from jax.experimental.pallas import tpu as pltpu
from jax.experimental.pallas import tpu_sc as plsc

VOCAB = 1000000
EMBED = 64
EMBEDP = 128
BATCH = 4096
SEQLEN = 200

NUM_CORES = 2
NUM_SUBCORES = 16
NW = NUM_CORES * NUM_SUBCORES
BPW = BATCH // NW
CHUNK0 = 104
CHUNK1 = SEQLEN - CHUNK0
NVREG = EMBED // 16

PREP_BLOCK = 32768


def _prep_body(tt_ref, out_ref):
    rows = lax.broadcasted_iota(jnp.int32, (EMBED, EMBEDP), 0)
    cols = lax.broadcasted_iota(jnp.int32, (EMBED, EMBEDP), 1)
    ident = (rows == cols).astype(jnp.float32)
    out_ref[...] = lax.dot_general(
        tt_ref[...], ident,
        dimension_numbers=(((0,), (0,)), ((), ())),
        preferred_element_type=jnp.float32,
    )


def _prep(tt):
    grid = pl.cdiv(VOCAB, PREP_BLOCK)
    return pl.pallas_call(
        _prep_body,
        grid=(grid,),
        in_specs=[pl.BlockSpec((EMBED, PREP_BLOCK), lambda i: (0, i))],
        out_specs=pl.BlockSpec((PREP_BLOCK, EMBEDP), lambda i: (i, 0)),
        out_shape=jax.ShapeDtypeStruct((VOCAB, EMBEDP), jnp.float32),
    )(tt)


def _bow_body(idx_hbm, table_hbm, out_hbm, idx_v, buf0, buf1, out_v,
              sem0, sem1):
    wid = lax.axis_index("s") * NUM_CORES + lax.axis_index("c")
    base = wid * BPW * SEQLEN

    pltpu.sync_copy(idx_hbm.at[pl.ds(base, BPW * SEQLEN)], idx_v)

    def fire(b, buf, sem):
        pltpu.async_copy(
            table_hbm.at[idx_v.at[pl.ds(b * SEQLEN, CHUNK0)]],
            buf.at[pl.ds(0, CHUNK0)], sem)
        pltpu.async_copy(
            table_hbm.at[idx_v.at[pl.ds(b * SEQLEN + CHUNK0, CHUNK1)]],
            buf.at[pl.ds(CHUNK0, CHUNK1)], sem)

    def drain(buf, sem):
        pltpu.make_async_copy(
            table_hbm.at[pl.ds(0, SEQLEN)], buf, sem).wait()

    def reduce_row(b, buf):
        accs = tuple(buf[0, pl.ds(16 * c, 16)] for c in range(NVREG))

        def red_body(r, accs):
            return tuple(
                jnp.maximum(a, buf[r, pl.ds(16 * c, 16)])
                for c, a in enumerate(accs))

        accs = lax.fori_loop(1, SEQLEN, red_body, accs, unroll=8)
        for c in range(NVREG):
            out_v[pl.ds(b * EMBED + 16 * c, 16)] = accs[c]

    fire(0, buf0, sem0)

    def pair_body(t, _):
        fire(2 * t + 1, buf1, sem1)
        drain(buf0, sem0)
        reduce_row(2 * t, buf0)

        @pl.when(t < BPW // 2 - 1)
        def _():
            fire(2 * t + 2, buf0, sem0)

        drain(buf1, sem1)
        reduce_row(2 * t + 1, buf1)
        return ()

    lax.fori_loop(0, BPW // 2, pair_body, ())

    pltpu.sync_copy(out_v, out_hbm.at[pl.ds(wid * BPW * EMBED, BPW * EMBED)])


@functools.cache
def _bow():
    return functools.partial(
        pl.kernel,
        mesh=plsc.VectorSubcoreMesh(core_axis_name="c", subcore_axis_name="s"),
        out_type=jax.ShapeDtypeStruct((BATCH * EMBED,), jnp.float32),
        scratch_types=[
            pltpu.VMEM((BPW * SEQLEN,), jnp.int32),
            pltpu.VMEM((SEQLEN, EMBED), jnp.float32),
            pltpu.VMEM((SEQLEN, EMBED), jnp.float32),
            pltpu.VMEM((BPW * EMBED,), jnp.float32),
            pltpu.SemaphoreType.DMA,
            pltpu.SemaphoreType.DMA,
        ],
        compiler_params=pltpu.CompilerParams(use_tc_tiling_on_sc=False),
    )(_bow_body)


@jax.jit
def kernel(input, table):
    tablep = _prep(table.T).reshape(2 * VOCAB, EMBED)
    idx = (input * 2).reshape(BATCH * SEQLEN)
    out = _bow()(idx, tablep)
    return out.reshape(BATCH, EMBED)


# speedup vs baseline: 2.6354x; 1.1990x over previous
import functools

import jax
import jax.numpy as jnp
from jax import lax
from jax.experimental import pallas as pl
from jax.experimental.pallas import tpu as pltpu
from jax.experimental.pallas import tpu_sc as plsc

VOCAB = 1000000
EMBED = 64
EMBEDP = 128
BATCH = 4096
SEQLEN = 200

NUM_CORES = 2
NUM_SUBCORES = 16
NW = NUM_CORES * NUM_SUBCORES
BPW = BATCH // NW
CHUNK0 = 104
CHUNK1 = SEQLEN - CHUNK0
NVREG = EMBED // 16

PREP_BLOCK = 16384
PREP_LOG2 = 14


def _prep_body(lo_ref, hi_ref, out_ref):
    rows = lax.broadcasted_iota(jnp.int32, (EMBED, EMBEDP), 0)
    cols = lax.broadcasted_iota(jnp.int32, (EMBED, EMBEDP), 1)
    il = (rows == cols).astype(jnp.float32)
    ir = (rows + EMBED == cols).astype(jnp.float32)
    out_ref[...] = lax.dot_general(
        lo_ref[...], il,
        dimension_numbers=(((0,), (0,)), ((), ())),
        preferred_element_type=jnp.float32,
    ) + lax.dot_general(
        hi_ref[...], ir,
        dimension_numbers=(((0,), (0,)), ((), ())),
        preferred_element_type=jnp.float32,
    )


def _prep(tt):
    grid = pl.cdiv(VOCAB, 2 * PREP_BLOCK)
    return pl.pallas_call(
        _prep_body,
        grid=(grid,),
        in_specs=[
            pl.BlockSpec((EMBED, PREP_BLOCK), lambda i: (0, 2 * i)),
            pl.BlockSpec((EMBED, PREP_BLOCK), lambda i: (0, 2 * i + 1)),
        ],
        out_specs=pl.BlockSpec((PREP_BLOCK, EMBEDP), lambda i: (i, 0)),
        out_shape=jax.ShapeDtypeStruct((grid * PREP_BLOCK, EMBEDP),
                                       jnp.float32),
    )(tt, tt)


def _bow_body(idx_hbm, table_hbm, out_hbm, idx_v, buf0, buf1, out_v,
              sem0, sem1):
    wid = lax.axis_index("s") * NUM_CORES + lax.axis_index("c")
    base = wid * BPW * SEQLEN

    pltpu.sync_copy(idx_hbm.at[pl.ds(base, BPW * SEQLEN)], idx_v)

    def fire(b, buf, sem):
        pltpu.async_copy(
            table_hbm.at[idx_v.at[pl.ds(b * SEQLEN, CHUNK0)]],
            buf.at[pl.ds(0, CHUNK0)], sem)
        pltpu.async_copy(
            table_hbm.at[idx_v.at[pl.ds(b * SEQLEN + CHUNK0, CHUNK1)]],
            buf.at[pl.ds(CHUNK0, CHUNK1)], sem)

    def drain(buf, sem):
        pltpu.make_async_copy(
            table_hbm.at[pl.ds(0, SEQLEN)], buf, sem).wait()

    def reduce_row(b, buf):
        accs = tuple(buf[0, pl.ds(16 * c, 16)] for c in range(NVREG))

        def red_body(r, accs):
            return tuple(
                jnp.maximum(a, buf[r, pl.ds(16 * c, 16)])
                for c, a in enumerate(accs))

        accs = lax.fori_loop(1, SEQLEN, red_body, accs, unroll=8)
        for c in range(NVREG):
            out_v[pl.ds(b * EMBED + 16 * c, 16)] = accs[c]

    fire(0, buf0, sem0)

    def pair_body(t, _):
        fire(2 * t + 1, buf1, sem1)
        drain(buf0, sem0)
        reduce_row(2 * t, buf0)

        @pl.when(t < BPW // 2 - 1)
        def _():
            fire(2 * t + 2, buf0, sem0)

        drain(buf1, sem1)
        reduce_row(2 * t + 1, buf1)
        return ()

    lax.fori_loop(0, BPW // 2, pair_body, ())

    pltpu.sync_copy(out_v, out_hbm.at[pl.ds(wid * BPW * EMBED, BPW * EMBED)])


@functools.cache
def _bow():
    return functools.partial(
        pl.kernel,
        mesh=plsc.VectorSubcoreMesh(core_axis_name="c", subcore_axis_name="s"),
        out_type=jax.ShapeDtypeStruct((BATCH * EMBED,), jnp.float32),
        scratch_types=[
            pltpu.VMEM((BPW * SEQLEN,), jnp.int32),
            pltpu.VMEM((SEQLEN, EMBED), jnp.float32),
            pltpu.VMEM((SEQLEN, EMBED), jnp.float32),
            pltpu.VMEM((BPW * EMBED,), jnp.float32),
            pltpu.SemaphoreType.DMA,
            pltpu.SemaphoreType.DMA,
        ],
        compiler_params=pltpu.CompilerParams(use_tc_tiling_on_sc=False),
    )(_bow_body)


@jax.jit
def kernel(input, table):
    tablep = _prep(table.T)
    tablev = tablep.reshape(tablep.shape[0] * 2, EMBED)
    g = input >> PREP_LOG2
    idx = (((g >> 1) << (PREP_LOG2 + 1))
           + ((input & (PREP_BLOCK - 1)) << 1)
           + (g & 1)).reshape(BATCH * SEQLEN)
    out = _bow()(idx, tablev)
    return out.reshape(BATCH, EMBED)
